# baseline (speedup 1.0000x reference)
import functools

import jax
import jax.numpy as jnp
from jax import lax
from jax.experimental import pallas as pl
from jax.experimental.pallas import tpu as pltpu
from jax.experimental.pallas import tpu_sc as plsc

T = 4
N = 10000
E = 320000
D = 128
NP = 10240
BLK = 1024
NB = NP // BLK

NWORK = 32
ECH = 128
CPW = 80
EP = NWORK * CPW * ECH
EROWS = EP // ECH


def _sn_body(w_ref, out_ref):
    W = w_ref[0]
    u = jnp.full((1, D), 1.0 / jnp.sqrt(jnp.float32(D)), jnp.float32)

    def body(_, carry):
        u, q = carry
        v = jnp.dot(u, W, preferred_element_type=jnp.float32)
        v = v / (jnp.sqrt(jnp.sum(v * v)) + 1e-12)
        q = lax.dot_general(v, W, (((1,), (1,)), ((), ())),
                            preferred_element_type=jnp.float32)
        u = q / (jnp.sqrt(jnp.sum(q * q)) + 1e-12)
        return (u, q)

    u, q = lax.fori_loop(0, 20, body, (u, jnp.zeros((1, D), jnp.float32)))
    sigma = jnp.sum(u * q)
    out_ref[0] = W / sigma


def _spectral_normalize_all(gc_W):
    return pl.pallas_call(
        _sn_body,
        grid=(T,),
        in_specs=[pl.BlockSpec((1, D, D), lambda t: (t, 0, 0))],
        out_specs=pl.BlockSpec((1, D, D), lambda t: (t, 0, 0)),
        out_shape=jax.ShapeDtypeStruct((T, D, D), jnp.float32),
    )(gc_W)


def _pre_body(x_ref, wphi_ref, bphi_ref, wsn_ref, phi_ref, xw_ref):
    x = x_ref[0]
    phi = jax.nn.relu(jnp.dot(x, wphi_ref[...],
                              preferred_element_type=jnp.float32) + bphi_ref[...])
    phi_ref[0] = phi
    xw_ref[0] = jnp.dot(phi, wsn_ref[0], preferred_element_type=jnp.float32)


def _pre(x_pad, W_phi, b_phi, W_sn):
    return pl.pallas_call(
        _pre_body,
        grid=(T, NB),
        in_specs=[
            pl.BlockSpec((1, BLK, D), lambda t, b: (t, b, 0)),
            pl.BlockSpec((D, D), lambda t, b: (0, 0)),
            pl.BlockSpec((1, D), lambda t, b: (0, 0)),
            pl.BlockSpec((1, D, D), lambda t, b: (t, 0, 0)),
        ],
        out_specs=[
            pl.BlockSpec((1, BLK, D), lambda t, b: (t, b, 0)),
            pl.BlockSpec((1, BLK, D), lambda t, b: (t, b, 0)),
        ],
        out_shape=[
            jax.ShapeDtypeStruct((T, NP, D), jnp.float32),
            jax.ShapeDtypeStruct((T, NP, D), jnp.float32),
        ],
    )(x_pad, W_phi, b_phi, W_sn)


def _ew_body(x_ref, dst_ref, o_ref, do_ref):
    o_ref[...] = jax.nn.sigmoid(x_ref[...])
    off = lax.broadcasted_iota(jnp.int32, (T, 1), 0) * NP
    do_ref[...] = dst_ref[...] + off


def _edge_sigmoid(edge_wt, dst):
    return pl.pallas_call(
        _ew_body,
        out_shape=[
            jax.ShapeDtypeStruct((T, E), jnp.float32),
            jax.ShapeDtypeStruct((T, E), jnp.int32),
        ],
    )(edge_wt, dst)


def _prep2_body(degp_ref, xw_ref, dinv_ref, xws_ref):
    deg = degp_ref[0, 0] + degp_ref[1, 0] + 1.0
    dinv = lax.rsqrt(deg)
    dinv_ref[0] = dinv
    xws_ref[0] = dinv * xw_ref[0]


def _prep2(degp, xw):
    return pl.pallas_call(
        _prep2_body,
        grid=(T, NB),
        in_specs=[
            pl.BlockSpec((2, 1, BLK, 1), lambda t, b: (0, t, b, 0)),
            pl.BlockSpec((1, BLK, D), lambda t, b: (t, b, 0)),
        ],
        out_specs=[
            pl.BlockSpec((1, BLK, 1), lambda t, b: (t, b, 0)),
            pl.BlockSpec((1, BLK, D), lambda t, b: (t, b, 0)),
        ],
        out_shape=[
            jax.ShapeDtypeStruct((T, NP, 1), jnp.float32),
            jax.ShapeDtypeStruct((T, NP, D), jnp.float32),
        ],
    )(degp, xw)


def _rep_body(acc_ref, xws_ref, dinv_ref, gcb_ref, rep_ref):
    acc = acc_ref[0, 0] + acc_ref[0, 1]
    out = dinv_ref[0] * (acc + xws_ref[0]) + gcb_ref[0]
    rep_ref[0] = jax.nn.relu(out)


def _rep(acc2, xws, dinv, gc_b):
    return pl.pallas_call(
        _rep_body,
        grid=(T, NB),
        in_specs=[
            pl.BlockSpec((1, 2, BLK, D), lambda t, b: (t, 0, b, 0)),
            pl.BlockSpec((1, BLK, D), lambda t, b: (t, b, 0)),
            pl.BlockSpec((1, BLK, 1), lambda t, b: (t, b, 0)),
            pl.BlockSpec((1, 1, D), lambda t, b: (t, 0, 0)),
        ],
        out_specs=pl.BlockSpec((1, BLK, D), lambda t, b: (t, b, 0)),
        out_shape=jax.ShapeDtypeStruct((T, NP, D), jnp.float32),
    )(acc2, xws, dinv, gc_b)


def _main_body(h_ref, rep_ref, phi_ref, c_ref, yh_ref,
               wfuse_ref, bfuse_ref, wih_ref, bih_ref, whh_ref, bhh_ref,
               w00_ref, b00_ref, w10_ref, b10_ref,
               w01_ref, b01_ref, w11_ref, b11_ref,
               psw1_ref, psb1_ref,
               z_ref, hnew_ref, y0_ref, y1_ref, q_ref, stats_ref):
    b = pl.program_id(0)
    h = h_ref[...]
    rep = rep_ref[...]
    phi = phi_ref[...]
    zin = jnp.concatenate([h, rep, phi], axis=1)
    z = jax.nn.relu(jnp.dot(zin, wfuse_ref[...],
                            preferred_element_type=jnp.float32) + bfuse_ref[...])
    z_ref[...] = z

    wih = wih_ref[...]
    gx = (jnp.dot(z, wih[0:D], preferred_element_type=jnp.float32)
          + c_ref[...] * wih[D:D + 1]
          + jnp.dot(yh_ref[...], wih[D + 1:D + 9],
                    preferred_element_type=jnp.float32)
          + bih_ref[...])
    gh = jnp.dot(h, whh_ref[...], preferred_element_type=jnp.float32) + bhh_ref[...]
    r = jax.nn.sigmoid(gx[:, 0:D] + gh[:, 0:D])
    u = jax.nn.sigmoid(gx[:, D:2 * D] + gh[:, D:2 * D])
    n = jnp.tanh(gx[:, 2 * D:3 * D] + r * gh[:, 2 * D:3 * D])
    hnew_ref[...] = (1.0 - u) * n + u * h

    y00 = jax.nn.relu(jnp.dot(z, w00_ref[...],
                              preferred_element_type=jnp.float32) + b00_ref[...])
    y0_ref[...] = jnp.dot(y00, w01_ref[...],
                          preferred_element_type=jnp.float32) + b01_ref[...]
    y10 = jax.nn.relu(jnp.dot(z, w10_ref[...],
                              preferred_element_type=jnp.float32) + b10_ref[...])
    y1_ref[...] = jnp.dot(y10, w11_ref[...],
                          preferred_element_type=jnp.float32) + b11_ref[...]

    q = jnp.dot(z, psw1_ref[...], preferred_element_type=jnp.float32) + psb1_ref[...]
    q_ref[...] = q
    rows = b * BLK + lax.broadcasted_iota(jnp.int32, (BLK, 1), 0)
    qm = jnp.where(rows < N, q, 0.0)
    s1 = jnp.sum(qm, axis=0, keepdims=True)
    s2 = jnp.sum(qm * qm, axis=0, keepdims=True)
    stats = jnp.concatenate([s1, s2], axis=0)

    @pl.when(b == 0)
    def _():
        stats_ref[...] = stats

    @pl.when(b != 0)
    def _():
        stats_ref[...] += stats


def _main_step(h, rep_t, phi_t, c_t, yh_t, p2):
    full = lambda shape: pl.BlockSpec(shape, lambda b: tuple(0 for _ in shape))
    return pl.pallas_call(
        _main_body,
        grid=(NB,),
        in_specs=[
            pl.BlockSpec((BLK, D), lambda b: (b, 0)),
            pl.BlockSpec((BLK, D), lambda b: (b, 0)),
            pl.BlockSpec((BLK, D), lambda b: (b, 0)),
            pl.BlockSpec((BLK, 1), lambda b: (b, 0)),
            pl.BlockSpec((BLK, 8), lambda b: (b, 0)),
            full((3 * D, D)), full((1, D)),
            full((D + 9, 3 * D)), full((1, 3 * D)),
            full((D, 3 * D)), full((1, 3 * D)),
            full((D, D)), full((1, D)),
            full((D, D)), full((1, D)),
            full((D, 1)), full((1, 1)),
            full((D, 1)), full((1, 1)),
            full((D, 100)), full((1, 100)),
        ],
        out_specs=[
            pl.BlockSpec((BLK, D), lambda b: (b, 0)),
            pl.BlockSpec((BLK, D), lambda b: (b, 0)),
            pl.BlockSpec((BLK, 1), lambda b: (b, 0)),
            pl.BlockSpec((BLK, 1), lambda b: (b, 0)),
            pl.BlockSpec((BLK, 100), lambda b: (b, 0)),
            pl.BlockSpec((2, 100), lambda b: (0, 0)),
        ],
        out_shape=[
            jax.ShapeDtypeStruct((NP, D), jnp.float32),
            jax.ShapeDtypeStruct((NP, D), jnp.float32),
            jax.ShapeDtypeStruct((NP, 1), jnp.float32),
            jax.ShapeDtypeStruct((NP, 1), jnp.float32),
            jax.ShapeDtypeStruct((NP, 100), jnp.float32),
            jax.ShapeDtypeStruct((2, 100), jnp.float32),
        ],
    )(h, rep_t, phi_t, c_t, yh_t,
      p2['W_fuse'], p2['b_fuse'], p2['W_ih'], p2['b_ih'], p2['W_hh'], p2['b_hh'],
      p2['W00'], p2['b00'], p2['W10'], p2['b10'],
      p2['W01'], p2['b01'], p2['W11'], p2['b11'],
      p2['ps_W1'], p2['ps_b1'])


def _ps_body(q_ref, stats_ref, gamma_ref, beta_ref, w2_ref, b2_ref, ps_ref):
    stats = stats_ref[0]
    mean = stats[0:1] * (1.0 / N)
    msq = stats[1:2] * (1.0 / N)
    var = msq - mean * mean
    qn = (q_ref[0] - mean) / jnp.sqrt(var + 1e-5) * gamma_ref[...] + beta_ref[...]
    s = jax.nn.sigmoid(qn)
    logits = jnp.dot(s, w2_ref[...], preferred_element_type=jnp.float32) + b2_ref[...]
    m = jnp.max(logits, axis=1, keepdims=True)
    e = jnp.exp(logits - m)
    ps_ref[0] = e / jnp.sum(e, axis=1, keepdims=True)


def _ps(q_all, stats_all, gamma, beta, w2, b2):
    return pl.pallas_call(
        _ps_body,
        grid=(T, NB),
        in_specs=[
            pl.BlockSpec((1, BLK, 100), lambda t, b: (t, b, 0)),
            pl.BlockSpec((1, 2, 100), lambda t, b: (t, 0, 0)),
            pl.BlockSpec((1, 100), lambda t, b: (0, 0)),
            pl.BlockSpec((1, 100), lambda t, b: (0, 0)),
            pl.BlockSpec((100, 2), lambda t, b: (0, 0)),
            pl.BlockSpec((1, 2), lambda t, b: (0, 0)),
        ],
        out_specs=pl.BlockSpec((1, BLK, 2), lambda t, b: (t, b, 0)),
        out_shape=jax.ShapeDtypeStruct((T, NP, 2), jnp.float32),
    )(q_all, stats_all, gamma, beta, w2, b2)


_SC_MESH = plsc.VectorSubcoreMesh(core_axis_name="c", subcore_axis_name="s")


_DEG_BATCH = 8


def _sc_deg_body(dst_h, ew_h, zd_h, out_h, idx_v, val_v, bounce_v, sem, deg_s):
    cid = lax.axis_index("c")
    sid = lax.axis_index("s")
    wid = cid * 16 + sid
    rpt = T * NP // 16
    r0 = sid * rpt
    pltpu.sync_copy(zd_h.at[pl.ds(r0, rpt)], bounce_v)
    pltpu.sync_copy(bounce_v, deg_s.at[pl.ds(r0, rpt)])
    for t in range(T):
        pltpu.sync_copy(dst_h.at[t, pl.ds(wid * CPW, CPW)],
                        idx_v.at[pl.ds(t * CPW, CPW)])
        pltpu.sync_copy(ew_h.at[t, pl.ds(wid * CPW, CPW)],
                        val_v.at[pl.ds(t * CPW, CPW)])
    plsc.subcore_barrier()

    def batch_body(bi, carry):
        descs = []
        for b in range(_DEG_BATCH):
            row = bi * _DEG_BATCH + b
            descs.append(pltpu.async_copy(
                val_v.at[row], deg_s.at[idx_v.at[row]], sem, add=True))
        for d in descs:
            d.wait()
        return carry

    lax.fori_loop(0, T * CPW // _DEG_BATCH, batch_body, 0)
    plsc.subcore_barrier()
    pltpu.sync_copy(deg_s.at[pl.ds(r0, rpt)], bounce_v)
    pltpu.sync_copy(bounce_v, out_h.at[cid, pl.ds(r0, rpt)])


def _sc_deg(dst3, ew3, zeros_d):
    return pl.kernel(
        _sc_deg_body,
        out_type=jax.ShapeDtypeStruct((2, T * NP), jnp.float32),
        mesh=_SC_MESH,
        scratch_types=[
            pltpu.VMEM((T * CPW, ECH), jnp.int32),
            pltpu.VMEM((T * CPW, ECH), jnp.float32),
            pltpu.VMEM((T * NP // 16,), jnp.float32),
            pltpu.SemaphoreType.DMA,
            pltpu.VMEM_SHARED((T * NP,), jnp.float32),
        ],
    )(dst3, ew3, zeros_d)


def _scale_rows(rows_v, ew_ref, ew_row):
    def group_body(g, c2):
        ewv = ew_ref[ew_row, pl.ds(g * 16, 16)]
        for k in range(16):
            r = g * 16 + k
            sv = jnp.full((16,), ewv[k], jnp.float32)
            for j in range(D // 16):
                rows_v[r, pl.ds(j * 16, 16)] = rows_v[r, pl.ds(j * 16, 16)] * sv
        return c2

    lax.fori_loop(0, ECH // 16, group_body, 0)


def _sc_agg_body(xws_h, src_h, dst_h, ew_h, za_h, out_h,
                 src_v, dstb, ewb, rows0, rows1, sem0, sem1, semi, acc_s):
    cid = lax.axis_index("c")
    sid = lax.axis_index("s")
    wid = cid * 16 + sid
    rpt = NP // 16
    r0 = sid * rpt
    pltpu.sync_copy(src_h.at[pl.ds(wid * CPW, CPW)], src_v)
    for b in range(rpt // ECH):
        pltpu.sync_copy(za_h.at[pl.ds(r0 + b * ECH, ECH)], rows0)
        pltpu.sync_copy(rows0, acc_s.at[pl.ds(r0 + b * ECH, ECH)])
    plsc.subcore_barrier()

    def pair_body(i, carry):
        ra = wid * CPW + 2 * i
        d_ga = pltpu.async_copy(xws_h.at[src_v.at[2 * i]], rows0, sem0)
        d_gb = pltpu.async_copy(xws_h.at[src_v.at[2 * i + 1]], rows1, sem1)
        d_da = pltpu.async_copy(dst_h.at[ra], dstb.at[0], semi)
        d_db = pltpu.async_copy(dst_h.at[ra + 1], dstb.at[1], semi)
        d_ea = pltpu.async_copy(ew_h.at[ra], ewb.at[0], semi)
        d_eb = pltpu.async_copy(ew_h.at[ra + 1], ewb.at[1], semi)
        d_da.wait()
        d_ea.wait()
        d_ga.wait()
        _scale_rows(rows0, ewb, 0)
        pltpu.sync_copy(rows0, acc_s.at[dstb.at[0]], add=True)
        d_db.wait()
        d_eb.wait()
        d_gb.wait()
        _scale_rows(rows1, ewb, 1)
        pltpu.sync_copy(rows1, acc_s.at[dstb.at[1]], add=True)
        return carry

    lax.fori_loop(0, CPW // 2, pair_body, 0)
    plsc.subcore_barrier()
    for b in range(rpt // ECH):
        pltpu.sync_copy(acc_s.at[pl.ds(r0 + b * ECH, ECH)], rows0)
        pltpu.sync_copy(rows0, out_h.at[cid, pl.ds(r0 + b * ECH, ECH)])


def _sc_agg(xws_t, src3_t, dst3_t, ew3_t, zeros_a):
    return pl.kernel(
        _sc_agg_body,
        out_type=jax.ShapeDtypeStruct((2, NP, D), jnp.float32),
        mesh=_SC_MESH,
        scratch_types=[
            pltpu.VMEM((CPW, ECH), jnp.int32),
            pltpu.VMEM((2, ECH), jnp.int32),
            pltpu.VMEM((2, ECH), jnp.float32),
            pltpu.VMEM((ECH, D), jnp.float32),
            pltpu.VMEM((ECH, D), jnp.float32),
            pltpu.SemaphoreType.DMA,
            pltpu.SemaphoreType.DMA,
            pltpu.SemaphoreType.DMA,
            pltpu.VMEM_SHARED((NP, D), jnp.float32),
        ],
    )(xws_t, src3_t, dst3_t, ew3_t, zeros_a)


def kernel(X_list, edge_index_list, C_list, Y_hist_list, params):
    p = params
    x_pad = jnp.pad(X_list, ((0, 0), (0, NP - N), (0, 0)))
    c_pad = jnp.pad(C_list, ((0, 0), (0, NP - N), (0, 0)))
    yh_pad = jnp.pad(Y_hist_list, ((0, 0), (0, NP - N), (0, 0)))
    src = edge_index_list[:, 0, :]
    dst = edge_index_list[:, 1, :]

    W_sn = _spectral_normalize_all(p['gc_W'])
    phi, xw = _pre(x_pad, p['W_phi'], p['b_phi'].reshape(1, D), W_sn)
    ew, dsto = _edge_sigmoid(p['edge_wt'], dst)

    src3 = jnp.pad(src, ((0, 0), (0, EP - E))).reshape(T, EROWS, ECH)
    dst3 = jnp.pad(dst, ((0, 0), (0, EP - E))).reshape(T, EROWS, ECH)
    dsto3 = jnp.pad(dsto, ((0, 0), (0, EP - E))).reshape(T, EROWS, ECH)
    ew_p = jnp.pad(ew, ((0, 0), (0, EP - E)))
    ew3 = ew_p.reshape(T, EROWS, ECH)
    zeros_d = jnp.zeros((T * NP,), jnp.float32)
    zeros_a = jnp.zeros((NP, D), jnp.float32)

    degp = _sc_deg(dsto3, ew3, zeros_d).reshape(2, T, NP, 1)
    dinv, xws = _prep2(degp, xw)
    acc2 = jnp.stack([_sc_agg(xws[t], src3[t], dst3[t], ew3[t], zeros_a)
                      for t in range(T)])
    rep = _rep(acc2, xws, dinv, p['gc_b'].reshape(T, 1, D))

    p2 = {
        'W_fuse': p['W_fuse'], 'b_fuse': p['b_fuse'].reshape(1, D),
        'W_ih': p['W_ih'], 'b_ih': p['b_ih'].reshape(1, 3 * D),
        'W_hh': p['W_hh'], 'b_hh': p['b_hh'].reshape(1, 3 * D),
        'W00': p['W00'], 'b00': p['b00'].reshape(1, D),
        'W10': p['W10'], 'b10': p['b10'].reshape(1, D),
        'W01': p['W01'], 'b01': p['b01'].reshape(1, 1),
        'W11': p['W11'], 'b11': p['b11'].reshape(1, 1),
        'ps_W1': p['ps_W1'], 'ps_b1': p['ps_b1'].reshape(1, 100),
    }

    h = jnp.zeros((NP, D), jnp.float32)
    zs, y0s, y1s, qs, stats = [], [], [], [], []
    for t in range(T):
        z, h, y0, y1, q, st = _main_step(h, rep[t], phi[t], c_pad[t], yh_pad[t], p2)
        zs.append(z)
        y0s.append(y0)
        y1s.append(y1)
        qs.append(q)
        stats.append(st)

    q_all = jnp.stack(qs)
    stats_all = jnp.stack(stats)
    ps = _ps(q_all, stats_all, p['bn_gamma'].reshape(1, 100),
             p['bn_beta'].reshape(1, 100), p['ps_W2'], p['ps_b2'].reshape(1, 2))

    y1_out = jnp.stack(y1s)[:, :N]
    y0_out = jnp.stack(y0s)[:, :N]
    z_out = jnp.stack(zs)[:, :N]
    ps_out = ps[:, :N]
    return (y1_out, y0_out, z_out, ps_out, h[:N])


# speedup vs baseline: 1.0375x; 1.0375x over previous
import functools

import jax
import jax.numpy as jnp
from jax import lax
from jax.experimental import pallas as pl
from jax.experimental.pallas import tpu as pltpu
from jax.experimental.pallas import tpu_sc as plsc

T = 4
N = 10000
E = 320000
D = 128
NP = 10240
BLK = 1024
NB = NP // BLK

NWORK = 32
ECH = 128
CPW = 80
EP = NWORK * CPW * ECH
EROWS = EP // ECH


def _sn_body(w_ref, out_ref):
    W = w_ref[0]
    u = jnp.full((1, D), 1.0 / jnp.sqrt(jnp.float32(D)), jnp.float32)

    def body(_, carry):
        u, q = carry
        v = jnp.dot(u, W, preferred_element_type=jnp.float32)
        v = v / (jnp.sqrt(jnp.sum(v * v)) + 1e-12)
        q = lax.dot_general(v, W, (((1,), (1,)), ((), ())),
                            preferred_element_type=jnp.float32)
        u = q / (jnp.sqrt(jnp.sum(q * q)) + 1e-12)
        return (u, q)

    u, q = lax.fori_loop(0, 20, body, (u, jnp.zeros((1, D), jnp.float32)))
    sigma = jnp.sum(u * q)
    out_ref[0] = W / sigma


def _spectral_normalize_all(gc_W):
    return pl.pallas_call(
        _sn_body,
        grid=(T,),
        in_specs=[pl.BlockSpec((1, D, D), lambda t: (t, 0, 0))],
        out_specs=pl.BlockSpec((1, D, D), lambda t: (t, 0, 0)),
        out_shape=jax.ShapeDtypeStruct((T, D, D), jnp.float32),
    )(gc_W)


def _pre_body(x_ref, wphi_ref, bphi_ref, wsn_ref, phi_ref, xw_ref):
    x = x_ref[0]
    phi = jax.nn.relu(jnp.dot(x, wphi_ref[...],
                              preferred_element_type=jnp.float32) + bphi_ref[...])
    phi_ref[0] = phi
    xw_ref[0] = jnp.dot(phi, wsn_ref[0], preferred_element_type=jnp.float32)


def _pre(x_pad, W_phi, b_phi, W_sn):
    return pl.pallas_call(
        _pre_body,
        grid=(T, NB),
        in_specs=[
            pl.BlockSpec((1, BLK, D), lambda t, b: (t, b, 0)),
            pl.BlockSpec((D, D), lambda t, b: (0, 0)),
            pl.BlockSpec((1, D), lambda t, b: (0, 0)),
            pl.BlockSpec((1, D, D), lambda t, b: (t, 0, 0)),
        ],
        out_specs=[
            pl.BlockSpec((1, BLK, D), lambda t, b: (t, b, 0)),
            pl.BlockSpec((1, BLK, D), lambda t, b: (t, b, 0)),
        ],
        out_shape=[
            jax.ShapeDtypeStruct((T, NP, D), jnp.float32),
            jax.ShapeDtypeStruct((T, NP, D), jnp.float32),
        ],
    )(x_pad, W_phi, b_phi, W_sn)


def _ew_body(x_ref, dst_ref, src_ref, o_ref, do_ref, so_ref):
    o_ref[...] = jax.nn.sigmoid(x_ref[...])
    off = lax.broadcasted_iota(jnp.int32, (T, 1), 0) * NP
    do_ref[...] = dst_ref[...] + off
    so_ref[...] = src_ref[...] + off


def _edge_sigmoid(edge_wt, dst, src):
    return pl.pallas_call(
        _ew_body,
        out_shape=[
            jax.ShapeDtypeStruct((T, E), jnp.float32),
            jax.ShapeDtypeStruct((T, E), jnp.int32),
            jax.ShapeDtypeStruct((T, E), jnp.int32),
        ],
    )(edge_wt, dst, src)


def _prep2_body(degp_ref, xw_ref, dinv_ref, xws_ref):
    deg = degp_ref[0, 0] + degp_ref[1, 0] + 1.0
    dinv = lax.rsqrt(deg)
    dinv_ref[0] = dinv
    xws_ref[0] = dinv * xw_ref[0]


def _prep2(degp, xw):
    return pl.pallas_call(
        _prep2_body,
        grid=(T, NB),
        in_specs=[
            pl.BlockSpec((2, 1, BLK, 1), lambda t, b: (0, t, b, 0)),
            pl.BlockSpec((1, BLK, D), lambda t, b: (t, b, 0)),
        ],
        out_specs=[
            pl.BlockSpec((1, BLK, 1), lambda t, b: (t, b, 0)),
            pl.BlockSpec((1, BLK, D), lambda t, b: (t, b, 0)),
        ],
        out_shape=[
            jax.ShapeDtypeStruct((T, NP, 1), jnp.float32),
            jax.ShapeDtypeStruct((T, NP, D), jnp.float32),
        ],
    )(degp, xw)


def _rep_body(acc_ref, xws_ref, dinv_ref, gcb_ref, rep_ref):
    acc = acc_ref[0, 0] + acc_ref[1, 0]
    out = dinv_ref[0] * (acc + xws_ref[0]) + gcb_ref[0]
    rep_ref[0] = jax.nn.relu(out)


def _rep(acc2, xws, dinv, gc_b):
    return pl.pallas_call(
        _rep_body,
        grid=(T, NB),
        in_specs=[
            pl.BlockSpec((2, 1, BLK, D), lambda t, b: (0, t, b, 0)),
            pl.BlockSpec((1, BLK, D), lambda t, b: (t, b, 0)),
            pl.BlockSpec((1, BLK, 1), lambda t, b: (t, b, 0)),
            pl.BlockSpec((1, 1, D), lambda t, b: (t, 0, 0)),
        ],
        out_specs=pl.BlockSpec((1, BLK, D), lambda t, b: (t, b, 0)),
        out_shape=jax.ShapeDtypeStruct((T, NP, D), jnp.float32),
    )(acc2, xws, dinv, gc_b)


def _main_body(h_ref, rep_ref, phi_ref, c_ref, yh_ref,
               wfuse_ref, bfuse_ref, wih_ref, bih_ref, whh_ref, bhh_ref,
               w00_ref, b00_ref, w10_ref, b10_ref,
               w01_ref, b01_ref, w11_ref, b11_ref,
               psw1_ref, psb1_ref,
               z_ref, hnew_ref, y0_ref, y1_ref, q_ref, stats_ref):
    b = pl.program_id(0)
    h = h_ref[...]
    rep = rep_ref[...]
    phi = phi_ref[...]
    zin = jnp.concatenate([h, rep, phi], axis=1)
    z = jax.nn.relu(jnp.dot(zin, wfuse_ref[...],
                            preferred_element_type=jnp.float32) + bfuse_ref[...])
    z_ref[...] = z

    wih = wih_ref[...]
    gx = (jnp.dot(z, wih[0:D], preferred_element_type=jnp.float32)
          + c_ref[...] * wih[D:D + 1]
          + jnp.dot(yh_ref[...], wih[D + 1:D + 9],
                    preferred_element_type=jnp.float32)
          + bih_ref[...])
    gh = jnp.dot(h, whh_ref[...], preferred_element_type=jnp.float32) + bhh_ref[...]
    r = jax.nn.sigmoid(gx[:, 0:D] + gh[:, 0:D])
    u = jax.nn.sigmoid(gx[:, D:2 * D] + gh[:, D:2 * D])
    n = jnp.tanh(gx[:, 2 * D:3 * D] + r * gh[:, 2 * D:3 * D])
    hnew_ref[...] = (1.0 - u) * n + u * h

    y00 = jax.nn.relu(jnp.dot(z, w00_ref[...],
                              preferred_element_type=jnp.float32) + b00_ref[...])
    y0_ref[...] = jnp.dot(y00, w01_ref[...],
                          preferred_element_type=jnp.float32) + b01_ref[...]
    y10 = jax.nn.relu(jnp.dot(z, w10_ref[...],
                              preferred_element_type=jnp.float32) + b10_ref[...])
    y1_ref[...] = jnp.dot(y10, w11_ref[...],
                          preferred_element_type=jnp.float32) + b11_ref[...]

    q = jnp.dot(z, psw1_ref[...], preferred_element_type=jnp.float32) + psb1_ref[...]
    q_ref[...] = q
    rows = b * BLK + lax.broadcasted_iota(jnp.int32, (BLK, 1), 0)
    qm = jnp.where(rows < N, q, 0.0)
    s1 = jnp.sum(qm, axis=0, keepdims=True)
    s2 = jnp.sum(qm * qm, axis=0, keepdims=True)
    stats = jnp.concatenate([s1, s2], axis=0)

    @pl.when(b == 0)
    def _():
        stats_ref[...] = stats

    @pl.when(b != 0)
    def _():
        stats_ref[...] += stats


def _main_step(h, rep_t, phi_t, c_t, yh_t, p2):
    full = lambda shape: pl.BlockSpec(shape, lambda b: tuple(0 for _ in shape))
    return pl.pallas_call(
        _main_body,
        grid=(NB,),
        in_specs=[
            pl.BlockSpec((BLK, D), lambda b: (b, 0)),
            pl.BlockSpec((BLK, D), lambda b: (b, 0)),
            pl.BlockSpec((BLK, D), lambda b: (b, 0)),
            pl.BlockSpec((BLK, 1), lambda b: (b, 0)),
            pl.BlockSpec((BLK, 8), lambda b: (b, 0)),
            full((3 * D, D)), full((1, D)),
            full((D + 9, 3 * D)), full((1, 3 * D)),
            full((D, 3 * D)), full((1, 3 * D)),
            full((D, D)), full((1, D)),
            full((D, D)), full((1, D)),
            full((D, 1)), full((1, 1)),
            full((D, 1)), full((1, 1)),
            full((D, 100)), full((1, 100)),
        ],
        out_specs=[
            pl.BlockSpec((BLK, D), lambda b: (b, 0)),
            pl.BlockSpec((BLK, D), lambda b: (b, 0)),
            pl.BlockSpec((BLK, 1), lambda b: (b, 0)),
            pl.BlockSpec((BLK, 1), lambda b: (b, 0)),
            pl.BlockSpec((BLK, 100), lambda b: (b, 0)),
            pl.BlockSpec((2, 100), lambda b: (0, 0)),
        ],
        out_shape=[
            jax.ShapeDtypeStruct((NP, D), jnp.float32),
            jax.ShapeDtypeStruct((NP, D), jnp.float32),
            jax.ShapeDtypeStruct((NP, 1), jnp.float32),
            jax.ShapeDtypeStruct((NP, 1), jnp.float32),
            jax.ShapeDtypeStruct((NP, 100), jnp.float32),
            jax.ShapeDtypeStruct((2, 100), jnp.float32),
        ],
    )(h, rep_t, phi_t, c_t, yh_t,
      p2['W_fuse'], p2['b_fuse'], p2['W_ih'], p2['b_ih'], p2['W_hh'], p2['b_hh'],
      p2['W00'], p2['b00'], p2['W10'], p2['b10'],
      p2['W01'], p2['b01'], p2['W11'], p2['b11'],
      p2['ps_W1'], p2['ps_b1'])


def _ps_body(q_ref, stats_ref, gamma_ref, beta_ref, w2_ref, b2_ref, ps_ref):
    stats = stats_ref[0]
    mean = stats[0:1] * (1.0 / N)
    msq = stats[1:2] * (1.0 / N)
    var = msq - mean * mean
    qn = (q_ref[0] - mean) / jnp.sqrt(var + 1e-5) * gamma_ref[...] + beta_ref[...]
    s = jax.nn.sigmoid(qn)
    logits = jnp.dot(s, w2_ref[...], preferred_element_type=jnp.float32) + b2_ref[...]
    m = jnp.max(logits, axis=1, keepdims=True)
    e = jnp.exp(logits - m)
    ps_ref[0] = e / jnp.sum(e, axis=1, keepdims=True)


def _ps(q_all, stats_all, gamma, beta, w2, b2):
    return pl.pallas_call(
        _ps_body,
        grid=(T, NB),
        in_specs=[
            pl.BlockSpec((1, BLK, 100), lambda t, b: (t, b, 0)),
            pl.BlockSpec((1, 2, 100), lambda t, b: (t, 0, 0)),
            pl.BlockSpec((1, 100), lambda t, b: (0, 0)),
            pl.BlockSpec((1, 100), lambda t, b: (0, 0)),
            pl.BlockSpec((100, 2), lambda t, b: (0, 0)),
            pl.BlockSpec((1, 2), lambda t, b: (0, 0)),
        ],
        out_specs=pl.BlockSpec((1, BLK, 2), lambda t, b: (t, b, 0)),
        out_shape=jax.ShapeDtypeStruct((T, NP, 2), jnp.float32),
    )(q_all, stats_all, gamma, beta, w2, b2)


_SC_MESH = plsc.VectorSubcoreMesh(core_axis_name="c", subcore_axis_name="s")


_DEG_BATCH = 8


def _sc_deg_body(dst_h, ew_h, zd_h, out_h, idx_v, val_v, bounce_v, sem, deg_s):
    cid = lax.axis_index("c")
    sid = lax.axis_index("s")
    wid = cid * 16 + sid
    rpt = T * NP // 16
    r0 = sid * rpt
    pltpu.sync_copy(zd_h.at[pl.ds(r0, rpt)], bounce_v)
    pltpu.sync_copy(bounce_v, deg_s.at[pl.ds(r0, rpt)])
    for t in range(T):
        pltpu.sync_copy(dst_h.at[t, pl.ds(wid * CPW, CPW)],
                        idx_v.at[pl.ds(t * CPW, CPW)])
        pltpu.sync_copy(ew_h.at[t, pl.ds(wid * CPW, CPW)],
                        val_v.at[pl.ds(t * CPW, CPW)])
    plsc.subcore_barrier()

    def batch_body(bi, carry):
        descs = []
        for b in range(_DEG_BATCH):
            row = bi * _DEG_BATCH + b
            descs.append(pltpu.async_copy(
                val_v.at[row], deg_s.at[idx_v.at[row]], sem, add=True))
        for d in descs:
            d.wait()
        return carry

    lax.fori_loop(0, T * CPW // _DEG_BATCH, batch_body, 0)
    plsc.subcore_barrier()
    pltpu.sync_copy(deg_s.at[pl.ds(r0, rpt)], bounce_v)
    pltpu.sync_copy(bounce_v, out_h.at[cid, pl.ds(r0, rpt)])


def _sc_deg(dst3, ew3, zeros_d):
    return pl.kernel(
        _sc_deg_body,
        out_type=jax.ShapeDtypeStruct((2, T * NP), jnp.float32),
        mesh=_SC_MESH,
        scratch_types=[
            pltpu.VMEM((T * CPW, ECH), jnp.int32),
            pltpu.VMEM((T * CPW, ECH), jnp.float32),
            pltpu.VMEM((T * NP // 16,), jnp.float32),
            pltpu.SemaphoreType.DMA,
            pltpu.VMEM_SHARED((T * NP,), jnp.float32),
        ],
    )(dst3, ew3, zeros_d)


def _scale_rows(rows_v, ew_ref, ew_row):
    def group_body(g, c2):
        ewv = ew_ref[ew_row, pl.ds(g * 16, 16)]
        for k in range(16):
            r = g * 16 + k
            sv = jnp.full((16,), ewv[k], jnp.float32)
            for j in range(D // 16):
                rows_v[r, pl.ds(j * 16, 16)] = rows_v[r, pl.ds(j * 16, 16)] * sv
        return c2

    lax.fori_loop(0, ECH // 16, group_body, 0)


def _sc_agg_body(xws_h, src_h, dst_h, ew_h, za_h, out_h,
                 src_v, dstb, ewb, rows0, rows1, sem0, sem1, semi, acc_s):
    cid = lax.axis_index("c")
    sid = lax.axis_index("s")
    wid = cid * 16 + sid
    rpt = NP // 16
    r0 = sid * rpt
    for b in range(rpt // ECH):
        pltpu.sync_copy(za_h.at[pl.ds(r0 + b * ECH, ECH)], rows0)
        pltpu.sync_copy(rows0, acc_s.at[pl.ds(r0 + b * ECH, ECH)])

    for t in range(T):
        pltpu.sync_copy(src_h.at[t, pl.ds(wid * CPW, CPW)], src_v)
        plsc.subcore_barrier()

        def pair_body(i, carry, t=t):
            ra = wid * CPW + 2 * i
            d_ga = pltpu.async_copy(xws_h.at[src_v.at[2 * i]], rows0, sem0)
            d_gb = pltpu.async_copy(xws_h.at[src_v.at[2 * i + 1]], rows1, sem1)
            d_da = pltpu.async_copy(dst_h.at[t, ra], dstb.at[0], semi)
            d_db = pltpu.async_copy(dst_h.at[t, ra + 1], dstb.at[1], semi)
            d_ea = pltpu.async_copy(ew_h.at[t, ra], ewb.at[0], semi)
            d_eb = pltpu.async_copy(ew_h.at[t, ra + 1], ewb.at[1], semi)
            d_da.wait()
            d_ea.wait()
            d_ga.wait()
            _scale_rows(rows0, ewb, 0)
            pltpu.sync_copy(rows0, acc_s.at[dstb.at[0]], add=True)
            d_db.wait()
            d_eb.wait()
            d_gb.wait()
            _scale_rows(rows1, ewb, 1)
            pltpu.sync_copy(rows1, acc_s.at[dstb.at[1]], add=True)
            return carry

        lax.fori_loop(0, CPW // 2, pair_body, 0)
        plsc.subcore_barrier()
        for b in range(rpt // ECH):
            pltpu.sync_copy(acc_s.at[pl.ds(r0 + b * ECH, ECH)], rows0)
            pltpu.sync_copy(rows0, out_h.at[cid, t, pl.ds(r0 + b * ECH, ECH)])
        if t < T - 1:
            for b in range(rpt // ECH):
                pltpu.sync_copy(za_h.at[pl.ds(r0 + b * ECH, ECH)], rows0)
                pltpu.sync_copy(rows0, acc_s.at[pl.ds(r0 + b * ECH, ECH)])


def _sc_agg(xws_flat, srco3, dst3, ew3, zeros_a):
    return pl.kernel(
        _sc_agg_body,
        out_type=jax.ShapeDtypeStruct((2, T, NP, D), jnp.float32),
        mesh=_SC_MESH,
        scratch_types=[
            pltpu.VMEM((CPW, ECH), jnp.int32),
            pltpu.VMEM((2, ECH), jnp.int32),
            pltpu.VMEM((2, ECH), jnp.float32),
            pltpu.VMEM((ECH, D), jnp.float32),
            pltpu.VMEM((ECH, D), jnp.float32),
            pltpu.SemaphoreType.DMA,
            pltpu.SemaphoreType.DMA,
            pltpu.SemaphoreType.DMA,
            pltpu.VMEM_SHARED((NP, D), jnp.float32),
        ],
    )(xws_flat, srco3, dst3, ew3, zeros_a)


def kernel(X_list, edge_index_list, C_list, Y_hist_list, params):
    p = params
    x_pad = jnp.pad(X_list, ((0, 0), (0, NP - N), (0, 0)))
    c_pad = jnp.pad(C_list, ((0, 0), (0, NP - N), (0, 0)))
    yh_pad = jnp.pad(Y_hist_list, ((0, 0), (0, NP - N), (0, 0)))
    src = edge_index_list[:, 0, :]
    dst = edge_index_list[:, 1, :]

    W_sn = _spectral_normalize_all(p['gc_W'])
    phi, xw = _pre(x_pad, p['W_phi'], p['b_phi'].reshape(1, D), W_sn)
    ew, dsto, srco = _edge_sigmoid(p['edge_wt'], dst, src)

    srco3 = jnp.pad(srco, ((0, 0), (0, EP - E))).reshape(T, EROWS, ECH)
    dst3 = jnp.pad(dst, ((0, 0), (0, EP - E))).reshape(T, EROWS, ECH)
    dsto3 = jnp.pad(dsto, ((0, 0), (0, EP - E))).reshape(T, EROWS, ECH)
    ew_p = jnp.pad(ew, ((0, 0), (0, EP - E)))
    ew3 = ew_p.reshape(T, EROWS, ECH)
    zeros_d = jnp.zeros((T * NP,), jnp.float32)
    zeros_a = jnp.zeros((NP, D), jnp.float32)

    degp = _sc_deg(dsto3, ew3, zeros_d).reshape(2, T, NP, 1)
    dinv, xws = _prep2(degp, xw)
    acc2 = _sc_agg(xws.reshape(T * NP, D), srco3, dst3, ew3, zeros_a)
    rep = _rep(acc2, xws, dinv, p['gc_b'].reshape(T, 1, D))

    p2 = {
        'W_fuse': p['W_fuse'], 'b_fuse': p['b_fuse'].reshape(1, D),
        'W_ih': p['W_ih'], 'b_ih': p['b_ih'].reshape(1, 3 * D),
        'W_hh': p['W_hh'], 'b_hh': p['b_hh'].reshape(1, 3 * D),
        'W00': p['W00'], 'b00': p['b00'].reshape(1, D),
        'W10': p['W10'], 'b10': p['b10'].reshape(1, D),
        'W01': p['W01'], 'b01': p['b01'].reshape(1, 1),
        'W11': p['W11'], 'b11': p['b11'].reshape(1, 1),
        'ps_W1': p['ps_W1'], 'ps_b1': p['ps_b1'].reshape(1, 100),
    }

    h = jnp.zeros((NP, D), jnp.float32)
    zs, y0s, y1s, qs, stats = [], [], [], [], []
    for t in range(T):
        z, h, y0, y1, q, st = _main_step(h, rep[t], phi[t], c_pad[t], yh_pad[t], p2)
        zs.append(z)
        y0s.append(y0)
        y1s.append(y1)
        qs.append(q)
        stats.append(st)

    q_all = jnp.stack(qs)
    stats_all = jnp.stack(stats)
    ps = _ps(q_all, stats_all, p['bn_gamma'].reshape(1, 100),
             p['bn_beta'].reshape(1, 100), p['ps_W2'], p['ps_b2'].reshape(1, 2))

    y1_out = jnp.stack(y1s)[:, :N]
    y0_out = jnp.stack(y0s)[:, :N]
    z_out = jnp.stack(zs)[:, :N]
    ps_out = ps[:, :N]
    return (y1_out, y0_out, z_out, ps_out, h[:N])
